# Initial kernel scaffold; baseline (speedup 1.0000x reference)
import jax
import jax.numpy as jnp
from jax.experimental import pallas as pl


def kernel(x_T, weight_VxD):
    raise NotImplementedError("write your pallas kernel here")


# speedup vs baseline: 1.0230x; 1.0230x over previous
import functools

import jax
import jax.numpy as jnp
from jax import lax
from jax.experimental import pallas as pl
from jax.experimental.pallas import tpu as pltpu
from jax.experimental.pallas import tpu_sc as plsc

_NC = 2
_NS = 16
_NW = _NC * _NS
_CHUNK = 128


def _emb_lookup(idx_2d, weight_VxD, n_chunks_per_w):
    n_rows = idx_2d.shape[0] * _CHUNK
    D = weight_VxD.shape[1]
    per_w = n_chunks_per_w * _CHUNK

    mesh = plsc.VectorSubcoreMesh(core_axis_name="c", subcore_axis_name="s")

    @functools.partial(
        pl.kernel,
        mesh=mesh,
        out_type=jax.ShapeDtypeStruct((n_rows, D), jnp.float32),
        scratch_types=[
            pltpu.VMEM((n_chunks_per_w, _CHUNK), jnp.int32),
            pltpu.VMEM((_CHUNK, D), jnp.float32),
            pltpu.SemaphoreType.DMA,
        ],
        compiler_params=pltpu.CompilerParams(use_tc_tiling_on_sc=False),
    )
    def emb(idx_hbm, table_hbm, out_hbm, idx_v, buf, sem):
        wid = lax.axis_index("s") * _NC + lax.axis_index("c")
        chunk_base = wid * n_chunks_per_w
        row_base = wid * per_w
        pltpu.sync_copy(idx_hbm.at[pl.ds(chunk_base, n_chunks_per_w)], idx_v)

        def body(j, carry):
            pltpu.async_copy(table_hbm.at[idx_v.at[j]], buf, sem).wait()
            pltpu.sync_copy(buf, out_hbm.at[pl.ds(row_base + j * _CHUNK, _CHUNK)])
            return carry

        lax.fori_loop(0, n_chunks_per_w, body, 0)

    return emb(idx_2d, weight_VxD)


def kernel(x_T, weight_VxD):
    B, L = x_T.shape
    V, D = weight_VxD.shape
    N = B * L
    assert N % (_NW * _CHUNK) == 0
    n_chunks_per_w = N // (_NW * _CHUNK)
    idx_2d = x_T.astype(jnp.int32).reshape(N // _CHUNK, _CHUNK)
    out = _emb_lookup(idx_2d, weight_VxD, n_chunks_per_w)
    return out.reshape(B, L, D)


# speedup vs baseline: 1.1117x; 1.0868x over previous
import functools

import jax
import jax.numpy as jnp
from jax import lax
from jax.experimental import pallas as pl
from jax.experimental.pallas import tpu as pltpu
from jax.experimental.pallas import tpu_sc as plsc

_NC = 2
_NS = 16
_NW = _NC * _NS
_CHUNK = 128
_NBUF = 8
_K = 4


def _emb_lookup(idx_2d, weight_VxD, n_chunks_per_w):
    n_rows = idx_2d.shape[0] * _CHUNK
    D = weight_VxD.shape[1]
    per_w = n_chunks_per_w * _CHUNK
    T = n_chunks_per_w
    n_main = T - _K - (_NBUF - _K)
    assert n_main % _NBUF == 0, "main loop must be a whole number of rings"
    n_groups = n_main // _NBUF

    mesh = plsc.VectorSubcoreMesh(core_axis_name="c", subcore_axis_name="s")

    @functools.partial(
        pl.kernel,
        mesh=mesh,
        out_type=jax.ShapeDtypeStruct((n_rows, D), jnp.float32),
        scratch_types=[
            pltpu.VMEM((n_chunks_per_w, _CHUNK), jnp.int32),
            pltpu.VMEM((_NBUF, _CHUNK, D), jnp.float32),
            pltpu.SemaphoreType.DMA((_NBUF,)),
            pltpu.SemaphoreType.DMA((_NBUF,)),
        ],
        compiler_params=pltpu.CompilerParams(use_tc_tiling_on_sc=False),
    )
    def emb(idx_hbm, table_hbm, out_hbm, idx_v, bufs, gsem, wsem):
        wid = lax.axis_index("s") * _NC + lax.axis_index("c")
        chunk_base = wid * n_chunks_per_w
        row_base = wid * per_w
        pltpu.sync_copy(idx_hbm.at[pl.ds(chunk_base, n_chunks_per_w)], idx_v)

        def fire_gather(j, b):
            pltpu.make_async_copy(
                table_hbm.at[idx_v.at[j]], bufs.at[b], gsem.at[b]
            ).start()

        def wait_gather(j, b):
            pltpu.make_async_copy(
                table_hbm.at[idx_v.at[j]], bufs.at[b], gsem.at[b]
            ).wait()

        def fire_wb(j, b):
            pltpu.make_async_copy(
                bufs.at[b], out_hbm.at[pl.ds(row_base + j * _CHUNK, _CHUNK)],
                wsem.at[b],
            ).start()

        def wait_wb(j, b):
            pltpu.make_async_copy(
                bufs.at[b], out_hbm.at[pl.ds(row_base + j * _CHUNK, _CHUNK)],
                wsem.at[b],
            ).wait()

        for c in range(_K):
            fire_gather(c, c)
        for s in range(_NBUF - _K):
            wait_gather(s, s % _NBUF)
            fire_wb(s, s % _NBUF)
            fire_gather(s + _K, (s + _K) % _NBUF)

        def group(g, carry):
            for b in range(_NBUF):
                s = (_NBUF - _K) + g * _NBUF + b
                cb = (_NBUF - _K + b) % _NBUF
                gb = (b + _NBUF - _K + _K) % _NBUF
                wait_gather(s, cb)
                fire_wb(s, cb)
                wait_wb(s + _K - _NBUF, gb)
                fire_gather(s + _K, gb)
            return carry

        lax.fori_loop(0, n_groups, group, 0)

        for t in range(_K):
            s = T - _K + t
            wait_gather(s, s % _NBUF)
            fire_wb(s, s % _NBUF)
        for t in range(_NBUF):
            s = T - _NBUF + t
            wait_wb(s, s % _NBUF)

    return emb(idx_2d, weight_VxD)


def kernel(x_T, weight_VxD):
    B, L = x_T.shape
    V, D = weight_VxD.shape
    N = B * L
    assert N % (_NW * _CHUNK) == 0
    n_chunks_per_w = N // (_NW * _CHUNK)
    idx_2d = x_T.astype(jnp.int32).reshape(N // _CHUNK, _CHUNK)
    out = _emb_lookup(idx_2d, weight_VxD, n_chunks_per_w)
    return out.reshape(B, L, D)


# speedup vs baseline: 1.6442x; 1.4789x over previous
import functools

import jax
import jax.numpy as jnp
from jax import lax
from jax.experimental import pallas as pl
from jax.experimental.pallas import tpu as pltpu
from jax.experimental.pallas import tpu_sc as plsc

_NC = 2
_NS = 16
_NW = _NC * _NS
_CHUNK = 128
_NBUF = 8


def _emb_lookup(x_LxB, weight_VxD):
    L, B = x_LxB.shape
    D = weight_VxD.shape[1]
    b_per_w = B // _NW
    n_sub = b_per_w // _CHUNK
    T = L * n_sub
    assert (T - 2 * _NBUF) % _NBUF == 0
    n_groups = (T - 2 * _NBUF) // _NBUF

    mesh = plsc.VectorSubcoreMesh(core_axis_name="c", subcore_axis_name="s")

    @functools.partial(
        pl.kernel,
        mesh=mesh,
        out_type=jax.ShapeDtypeStruct((L, D, B), jnp.float32),
        scratch_types=[
            pltpu.VMEM((L, b_per_w), jnp.int32),
            pltpu.VMEM((_NBUF, _CHUNK, D), jnp.float32),
            pltpu.VMEM((_NBUF, D, _CHUNK), jnp.float32),
            pltpu.SemaphoreType.DMA((_NBUF,)),
            pltpu.SemaphoreType.DMA((_NBUF,)),
        ],
        compiler_params=pltpu.CompilerParams(
            use_tc_tiling_on_sc=False, needs_layout_passes=False
        ),
    )
    def emb(x_hbm, table_hbm, out_hbm, idx_v, gbufs, tbufs, gsem, wsem):
        wid = lax.axis_index("s") * _NC + lax.axis_index("c")
        bbase = wid * b_per_w

        def stage(l, c):
            pltpu.sync_copy(x_hbm.at[l, pl.ds(bbase, b_per_w)], idx_v.at[l])
            return c

        lax.fori_loop(0, L, stage, 0)

        iota_lo = lax.iota(jnp.int32, 16)
        iota_hi = iota_lo + 16

        def fire_gather(j, b):
            l, s = j // n_sub, j % n_sub
            pltpu.make_async_copy(
                table_hbm.at[idx_v.at[l, pl.ds(s * _CHUNK, _CHUNK)]],
                gbufs.at[b], gsem.at[b],
            ).start()

        def wait_gather(j, b):
            l, s = j // n_sub, j % n_sub
            pltpu.make_async_copy(
                table_hbm.at[idx_v.at[l, pl.ds(s * _CHUNK, _CHUNK)]],
                gbufs.at[b], gsem.at[b],
            ).wait()

        def shuffle(b, m):
            def row(r, c):
                v0 = gbufs[b, r, pl.ds(0, 16)]
                v1 = gbufs[b, r, pl.ds(16, 16)]
                rv = jnp.full((16,), r, jnp.int32)
                plsc.store_scatter(tbufs.at[m], [iota_lo, rv], v0)
                plsc.store_scatter(tbufs.at[m], [iota_hi, rv], v1)
                return c

            lax.fori_loop(0, _CHUNK, row, 0)

        def fire_wb(j, m):
            l, s = j // n_sub, j % n_sub
            pltpu.make_async_copy(
                tbufs.at[m],
                out_hbm.at[l, :, pl.ds(bbase + s * _CHUNK, _CHUNK)],
                wsem.at[m],
            ).start()

        def wait_wb(j, m):
            l, s = j // n_sub, j % n_sub
            pltpu.make_async_copy(
                tbufs.at[m],
                out_hbm.at[l, :, pl.ds(bbase + s * _CHUNK, _CHUNK)],
                wsem.at[m],
            ).wait()

        for c in range(_NBUF):
            fire_gather(c, c)
        for j in range(_NBUF):
            wait_gather(j, j)
            shuffle(j, j)
            fire_wb(j, j)
            fire_gather(j + _NBUF, j)

        def group(g, carry):
            for b in range(_NBUF):
                j = _NBUF + g * _NBUF + b
                wait_gather(j, b)
                wait_wb(j - _NBUF, b)
                shuffle(b, b)
                fire_wb(j, b)
                fire_gather(j + _NBUF, b)
            return carry

        lax.fori_loop(0, n_groups, group, 0)

        for t in range(_NBUF):
            j = T - _NBUF + t
            b = j % _NBUF
            wait_gather(j, b)
            wait_wb(j - _NBUF, b)
            shuffle(b, b)
            fire_wb(j, b)
        for t in range(_NBUF):
            j = T - _NBUF + t
            wait_wb(j, j % _NBUF)

    return emb(x_LxB, weight_VxD)


def kernel(x_T, weight_VxD):
    B, L = x_T.shape
    V, D = weight_VxD.shape
    x_LxB = x_T.T.astype(jnp.int32)
    out_LDB = _emb_lookup(x_LxB, weight_VxD)
    return out_LDB.transpose(2, 0, 1)
